# Initial kernel scaffold; baseline (speedup 1.0000x reference)
import jax
import jax.numpy as jnp
from jax.experimental import pallas as pl


def kernel(x, edge_index, e_features, params):
    raise NotImplementedError("write your pallas kernel here")


# speedup vs baseline: 1.4138x; 1.4138x over previous
import functools

import jax
import jax.numpy as jnp
from jax import lax
from jax.experimental import pallas as pl
from jax.experimental.pallas import tpu as pltpu
from jax.experimental.pallas import tpu_sc as plsc

_N = 50000
_E = 800000
_E_PAD = 802816
_NW = 32
_EPW = _E_PAD // _NW
_NIDX = _EPW // 128
_CHUNK = 512
_NCH = _EPW // _CHUNK
_SCCH = 256
_NSCCH = _EPW // _SCCH
_HALF = 25000
_ACC = 25024
_TROWS = _ACC // 16
_LAT = 64
_BLK_E = 1024
_BLK_N = 1000


def _ln(y, g, b):
    m = jnp.mean(y, axis=-1, keepdims=True)
    v = jnp.mean((y - m) ** 2, axis=-1, keepdims=True)
    return (y - m) / jnp.sqrt(v + 1e-5) * g + b


def _dot(a, w):
    return jnp.dot(a, w, preferred_element_type=jnp.float32)


def _enc_body(x_ref, w1, b1, w2, b2, w3, b3, g, bb, o_ref):
    y = jnp.maximum(_dot(x_ref[...], w1[...]) + b1[...], 0.0)
    y = jnp.maximum(_dot(y, w2[...]) + b2[...], 0.0)
    y = _dot(y, w3[...]) + b3[...]
    o_ref[...] = _ln(y, g[...], bb[...])


def _edge_body(xi, xj, e, w1a, w1b, w1c, b1, w2, b2, w3, b3, g, bb,
               m_o, e_o):
    ev = e[...]
    y = (_dot(xi[...], w1a[...]) + _dot(xj[...], w1b[...])
         + _dot(ev, w1c[...]) + b1[...])
    y = jnp.maximum(y, 0.0)
    y = jnp.maximum(_dot(y, w2[...]) + b2[...], 0.0)
    y = _dot(y, w3[...]) + b3[...]
    m = _ln(y, g[...], bb[...])
    m_o[...] = m
    e_o[...] = m + ev


def _node_body(pa, pb, h, w1a, w1b, b1, w2, b2, w3, b3, g, bb, o_ref):
    hv = h[...]
    agg = pa[0, 0] + pb[0, 0]
    y = _dot(agg, w1a[...]) + _dot(hv, w1b[...]) + b1[...]
    y = jnp.maximum(y, 0.0)
    y = jnp.maximum(_dot(y, w2[...]) + b2[...], 0.0)
    y = _dot(y, w3[...]) + b3[...]
    o_ref[...] = _ln(y, g[...], bb[...]) + hv


def _dec_body(h, w1, b1, w2, b2, w3, b3, o_ref):
    y = jnp.maximum(_dot(h[...], w1[...]) + b1[...], 0.0)
    y = jnp.maximum(_dot(y, w2[...]) + b2[...], 0.0)
    o_ref[...] = _dot(y, w3[...]) + b3[...]


def _full(shape):
    nd = len(shape)
    return pl.BlockSpec(shape, lambda i: (0,) * nd)


def _rows(blk, width):
    return pl.BlockSpec((blk, width), lambda i: (i, 0))


def _enc_call(xarr, wts, blk):
    n, d = xarr.shape
    return pl.pallas_call(
        _enc_body,
        grid=(n // blk,),
        in_specs=[_rows(blk, d)] + [_full(w.shape) for w in wts],
        out_specs=_rows(blk, _LAT),
        out_shape=jax.ShapeDtypeStruct((n, _LAT), jnp.float32),
    )(xarr, *wts)


def _edge_call(xi, xj, e, wts):
    return pl.pallas_call(
        _edge_body,
        grid=(_E_PAD // _BLK_E,),
        in_specs=[_rows(_BLK_E, _LAT)] * 3 + [_full(w.shape) for w in wts],
        out_specs=[_rows(_BLK_E, _LAT)] * 2,
        out_shape=[jax.ShapeDtypeStruct((_E_PAD, _LAT), jnp.float32)] * 2,
    )(xi, xj, e, *wts)


def _node_call(part, h, wts):
    nb = _HALF // _BLK_N
    pspec_a = pl.BlockSpec((1, 1, _BLK_N, _LAT),
                           lambda i: (0, i // nb, i % nb, 0))
    pspec_b = pl.BlockSpec((1, 1, _BLK_N, _LAT),
                           lambda i: (1, i // nb, i % nb, 0))
    return pl.pallas_call(
        _node_body,
        grid=(_N // _BLK_N,),
        in_specs=[pspec_a, pspec_b, _rows(_BLK_N, _LAT)]
        + [_full(w.shape) for w in wts],
        out_specs=_rows(_BLK_N, _LAT),
        out_shape=jax.ShapeDtypeStruct((_N, _LAT), jnp.float32),
    )(part, part, h, *wts)


def _dec_call(h, wts, out_dim):
    return pl.pallas_call(
        _dec_body,
        grid=(_N // _BLK_N,),
        in_specs=[_rows(_BLK_N, _LAT)] + [_full(w.shape) for w in wts],
        out_specs=_rows(_BLK_N, out_dim),
        out_shape=jax.ShapeDtypeStruct((_N, out_dim), jnp.float32),
    )(h, *wts)


@functools.cache
def _gather_kernel():
    mesh = plsc.VectorSubcoreMesh(core_axis_name="c", subcore_axis_name="s")

    @functools.partial(
        pl.kernel,
        out_type=[jax.ShapeDtypeStruct((_E_PAD, _LAT), jnp.float32),
                  jax.ShapeDtypeStruct((_E_PAD, _LAT), jnp.float32)],
        mesh=mesh,
        scratch_types=[pltpu.VMEM((_NIDX, 128), jnp.int32),
                       pltpu.VMEM((_CHUNK, _LAT), jnp.float32),
                       pltpu.SemaphoreType.DMA],
        compiler_params=pltpu.CompilerParams(use_tc_tiling_on_sc=False),
    )
    def gather_k(h_hbm, sidx_hbm, ridx_hbm, xj_hbm, xi_hbm,
                 idx_v, rows_v, sem):
        wid = lax.axis_index("s") * 2 + lax.axis_index("c")
        base = wid * _EPW

        def run(idx_hbm, out_hbm):
            pltpu.sync_copy(idx_hbm.at[wid], idx_v)

            @pl.loop(0, _NCH)
            def _chunk(i):
                cps = [
                    pltpu.async_copy(h_hbm.at[idx_v.at[i * 4 + jj]],
                                     rows_v.at[pl.ds(jj * 128, 128)], sem)
                    for jj in range(4)
                ]
                for cp in cps:
                    cp.wait()
                pltpu.sync_copy(rows_v,
                                out_hbm.at[pl.ds(base + i * _CHUNK, _CHUNK)])

        run(sidx_hbm, xj_hbm)
        run(ridx_hbm, xi_hbm)

    return gather_k


@functools.cache
def _scatter_kernel():
    mesh = plsc.VectorSubcoreMesh(core_axis_name="c", subcore_axis_name="s")

    @functools.partial(
        pl.kernel,
        out_type=jax.ShapeDtypeStruct((2, 2, _ACC, _LAT), jnp.float32),
        mesh=mesh,
        scratch_types=[pltpu.VMEM((2, 128), jnp.int32),
                       pltpu.VMEM((_SCCH, _LAT), jnp.float32),
                       pltpu.VMEM_SHARED((_ACC, _LAT), jnp.float32)],
        compiler_params=pltpu.CompilerParams(use_tc_tiling_on_sc=False),
    )
    def scatter_k(m_hbm, lidx_hbm, zeros_hbm, part_hbm, idx_v, rows_v, acc):
        c = lax.axis_index("c")
        s = lax.axis_index("s")
        wid = s * 2 + c
        base = wid * _EPW
        for p in range(2):
            pltpu.sync_copy(zeros_hbm, acc.at[pl.ds(s * _TROWS, _TROWS)])
            plsc.subcore_barrier()

            @pl.loop(0, _NSCCH)
            def _chunk(i):
                pltpu.sync_copy(
                    lidx_hbm.at[wid, pl.ds(p * _NIDX + i * 2, 2)], idx_v)
                pltpu.sync_copy(m_hbm.at[pl.ds(base + i * _SCCH, _SCCH)],
                                rows_v)
                for jj in range(2):
                    pltpu.sync_copy(
                        rows_v.at[pl.ds(jj * 128, 128)],
                        acc.at[idx_v.at[jj]],
                        add=True)

            plsc.subcore_barrier()
            pltpu.sync_copy(acc.at[pl.ds(s * _TROWS, _TROWS)],
                            part_hbm.at[c, p, pl.ds(s * _TROWS, _TROWS)])

    return scatter_k


def _mlp_wts(mlp):
    out = []
    for w, b in mlp:
        out.append(w)
        out.append(b.reshape(1, -1))
    return out


def kernel(x, edge_index, e_features, params):
    senders = edge_index[0]
    receivers = edge_index[1]
    pad = _E_PAD - _E
    sp = jnp.concatenate([senders, jnp.zeros((pad,), jnp.int32)])
    rp = jnp.concatenate([receivers, jnp.zeros((pad,), jnp.int32)])
    rs = jnp.concatenate([receivers, jnp.full((pad,), -1, jnp.int32)])
    lidx = jnp.concatenate(
        [jnp.where((rs >= p * _HALF) & (rs < (p + 1) * _HALF),
                   rs - p * _HALF, _HALF).reshape(_NW, _NIDX, 128)
         for p in range(2)],
        axis=1)
    sidx = sp.reshape(_NW, _NIDX, 128)
    ridx = rp.reshape(_NW, _NIDX, 128)
    e_pad = jnp.concatenate(
        [e_features, jnp.zeros((pad, e_features.shape[1]), jnp.float32)])
    zeros = jnp.zeros((_TROWS, _LAT), jnp.float32)

    g, b = params['enc_node_ln']
    enc_n_wts = _mlp_wts(params['enc_node_mlp']) + [g.reshape(1, -1),
                                                    b.reshape(1, -1)]
    g, b = params['enc_edge_ln']
    enc_e_wts = _mlp_wts(params['enc_edge_mlp']) + [g.reshape(1, -1),
                                                    b.reshape(1, -1)]

    h = _enc_call(x, enc_n_wts, _BLK_N)
    e = _enc_call(e_pad, enc_e_wts, _BLK_E)

    for layer in params['gnn']:
        xj, xi = _gather_kernel()(h, sidx, ridx)

        (w1, b1), (w2, b2), (w3, b3) = layer['edge_mlp']
        g, bb = layer['edge_ln']
        edge_wts = [w1[0:_LAT], w1[_LAT:2 * _LAT], w1[2 * _LAT:3 * _LAT],
                    b1.reshape(1, -1), w2, b2.reshape(1, -1),
                    w3, b3.reshape(1, -1),
                    g.reshape(1, -1), bb.reshape(1, -1)]
        m, e = _edge_call(xi, xj, e, edge_wts)

        part = _scatter_kernel()(m, lidx, zeros)

        (w1, b1), (w2, b2), (w3, b3) = layer['node_mlp']
        g, bb = layer['node_ln']
        node_wts = [w1[0:_LAT], w1[_LAT:2 * _LAT],
                    b1.reshape(1, -1), w2, b2.reshape(1, -1),
                    w3, b3.reshape(1, -1),
                    g.reshape(1, -1), bb.reshape(1, -1)]
        h = _node_call(part, h, node_wts)

    dec_wts = _mlp_wts(params['dec_mlp'])
    out_dim = params['dec_mlp'][-1][0].shape[1]
    return _dec_call(h, dec_wts, out_dim)


# speedup vs baseline: 1.9689x; 1.3926x over previous
import functools

import jax
import jax.numpy as jnp
from jax import lax
from jax.experimental import pallas as pl
from jax.experimental.pallas import tpu as pltpu
from jax.experimental.pallas import tpu_sc as plsc

_N = 50000
_E = 800000
_E_PAD = 802816
_NW = 32
_EPW = _E_PAD // _NW
_NIDX = _EPW // 128
_CHUNK = 512
_NCH = _EPW // _CHUNK
_SCCH = 256
_NSCCH = _EPW // _SCCH
_HALF = 25600
_ACC = 25632
_TROWS = _ACC // 16
_LAT = 64

_EP2 = _E_PAD // 2
_NP2 = _N // 2
_ACC2 = _ACC // 2
_BLK_E = 512
_BLK_N = 200


def _dot(a, w):
    return jnp.dot(a, w, preferred_element_type=jnp.float32)


def _ln_packed(y, g, b, mm):
    mu = _dot(y, mm)
    d = y - mu
    v = _dot(d * d, mm)
    return d / jnp.sqrt(v + 1e-5) * g + b


def _enc_body(x_ref, w1, b1, w2, b2, w3, b3, g, bb, mm, o_ref):
    y = jnp.maximum(_dot(x_ref[...], w1[...]) + b1[...], 0.0)
    y = jnp.maximum(_dot(y, w2[...]) + b2[...], 0.0)
    y = _dot(y, w3[...]) + b3[...]
    o_ref[...] = _ln_packed(y, g[...], bb[...], mm[...])


def _edge_body(xi, xj, e, w1a, w1b, w1c, b1, w2, b2, w3, b3, g, bb, mm,
               m_o, e_o):
    ev = e[...]
    y = (_dot(xi[...], w1a[...]) + _dot(xj[...], w1b[...])
         + _dot(ev, w1c[...]) + b1[...])
    y = jnp.maximum(y, 0.0)
    y = jnp.maximum(_dot(y, w2[...]) + b2[...], 0.0)
    y = _dot(y, w3[...]) + b3[...]
    m = _ln_packed(y, g[...], bb[...], mm[...])
    m_o[...] = m
    e_o[...] = m + ev


def _node_body(pa, pb, h, w1a, w1b, b1, w2, b2, w3, b3, g, bb, mm, o_ref):
    hv = h[...]
    agg = pa[0, 0] + pb[0, 0]
    y = _dot(agg, w1a[...]) + _dot(hv, w1b[...]) + b1[...]
    y = jnp.maximum(y, 0.0)
    y = jnp.maximum(_dot(y, w2[...]) + b2[...], 0.0)
    y = _dot(y, w3[...]) + b3[...]
    o_ref[...] = _ln_packed(y, g[...], bb[...], mm[...]) + hv


def _dec_body(h, w1, b1, w2, b2, w3, b3, o_ref):
    y = jnp.maximum(_dot(h[...], w1[...]) + b1[...], 0.0)
    y = jnp.maximum(_dot(y, w2[...]) + b2[...], 0.0)
    o_ref[...] = _dot(y, w3[...]) + b3[...]


def _full(shape):
    nd = len(shape)
    return pl.BlockSpec(shape, lambda i: (0,) * nd)


def _rows(blk, width):
    return pl.BlockSpec((blk, width), lambda i: (i, 0))


def _enc_call(xarr, wts, blk):
    n, d = xarr.shape
    return pl.pallas_call(
        _enc_body,
        grid=(n // blk,),
        in_specs=[_rows(blk, d)] + [_full(w.shape) for w in wts],
        out_specs=_rows(blk, 128),
        out_shape=jax.ShapeDtypeStruct((n, 128), jnp.float32),
    )(xarr, *wts)


def _edge_call(xi, xj, e, wts):
    return pl.pallas_call(
        _edge_body,
        grid=(_EP2 // _BLK_E,),
        in_specs=[_rows(_BLK_E, 128)] * 3 + [_full(w.shape) for w in wts],
        out_specs=[_rows(_BLK_E, 128)] * 2,
        out_shape=[jax.ShapeDtypeStruct((_EP2, 128), jnp.float32)] * 2,
    )(xi, xj, e, *wts)


def _node_call(part, h, wts):
    nb = _HALF // 2 // _BLK_N
    pspec_a = pl.BlockSpec((1, 1, _BLK_N, 128),
                           lambda i: (0, i // nb, i % nb, 0))
    pspec_b = pl.BlockSpec((1, 1, _BLK_N, 128),
                           lambda i: (1, i // nb, i % nb, 0))
    return pl.pallas_call(
        _node_body,
        grid=(_NP2 // _BLK_N,),
        in_specs=[pspec_a, pspec_b, _rows(_BLK_N, 128)]
        + [_full(w.shape) for w in wts],
        out_specs=_rows(_BLK_N, 128),
        out_shape=jax.ShapeDtypeStruct((_NP2, 128), jnp.float32),
    )(part, part, h, *wts)


def _dec_call(h, wts, out_dim):
    return pl.pallas_call(
        _dec_body,
        grid=(_NP2 // _BLK_N,),
        in_specs=[_rows(_BLK_N, 128)] + [_full(w.shape) for w in wts],
        out_specs=_rows(_BLK_N, 2 * out_dim),
        out_shape=jax.ShapeDtypeStruct((_NP2, 2 * out_dim), jnp.float32),
    )(h, *wts)


@functools.cache
def _gather_kernel():
    mesh = plsc.VectorSubcoreMesh(core_axis_name="c", subcore_axis_name="s")

    @functools.partial(
        pl.kernel,
        out_type=[jax.ShapeDtypeStruct((_E_PAD, _LAT), jnp.float32),
                  jax.ShapeDtypeStruct((_E_PAD, _LAT), jnp.float32)],
        mesh=mesh,
        scratch_types=[pltpu.VMEM((_NIDX, 128), jnp.int32),
                       pltpu.VMEM((_CHUNK, _LAT), jnp.float32),
                       pltpu.SemaphoreType.DMA],
        compiler_params=pltpu.CompilerParams(use_tc_tiling_on_sc=False),
    )
    def gather_k(h_hbm, sidx_hbm, ridx_hbm, xj_hbm, xi_hbm,
                 idx_v, rows_v, sem):
        wid = lax.axis_index("s") * 2 + lax.axis_index("c")
        base = wid * _EPW

        def run(idx_hbm, out_hbm):
            pltpu.sync_copy(idx_hbm.at[wid], idx_v)

            @pl.loop(0, _NCH)
            def _chunk(i):
                cps = [
                    pltpu.async_copy(h_hbm.at[idx_v.at[i * 4 + jj]],
                                     rows_v.at[pl.ds(jj * 128, 128)], sem)
                    for jj in range(4)
                ]
                for cp in cps:
                    cp.wait()
                pltpu.sync_copy(rows_v,
                                out_hbm.at[pl.ds(base + i * _CHUNK, _CHUNK)])

        run(sidx_hbm, xj_hbm)
        run(ridx_hbm, xi_hbm)

    return gather_k


@functools.cache
def _scatter_kernel():
    mesh = plsc.VectorSubcoreMesh(core_axis_name="c", subcore_axis_name="s")

    @functools.partial(
        pl.kernel,
        out_type=jax.ShapeDtypeStruct((2, 2, _ACC, _LAT), jnp.float32),
        mesh=mesh,
        scratch_types=[pltpu.VMEM((2, 128), jnp.int32),
                       pltpu.VMEM((_SCCH, _LAT), jnp.float32),
                       pltpu.VMEM_SHARED((_ACC, _LAT), jnp.float32)],
        compiler_params=pltpu.CompilerParams(use_tc_tiling_on_sc=False),
    )
    def scatter_k(m_hbm, lidx_hbm, zeros_hbm, part_hbm, idx_v, rows_v, acc):
        c = lax.axis_index("c")
        s = lax.axis_index("s")
        wid = s * 2 + c
        base = wid * _EPW
        for p in range(2):
            pltpu.sync_copy(zeros_hbm, acc.at[pl.ds(s * _TROWS, _TROWS)])
            plsc.subcore_barrier()

            @pl.loop(0, _NSCCH)
            def _chunk(i):
                pltpu.sync_copy(
                    lidx_hbm.at[wid, pl.ds(p * _NIDX + i * 2, 2)], idx_v)
                pltpu.sync_copy(m_hbm.at[pl.ds(base + i * _SCCH, _SCCH)],
                                rows_v)
                for jj in range(2):
                    pltpu.sync_copy(
                        rows_v.at[pl.ds(jj * 128, 128)],
                        acc.at[idx_v.at[jj]],
                        add=True)

            plsc.subcore_barrier()
            pltpu.sync_copy(acc.at[pl.ds(s * _TROWS, _TROWS)],
                            part_hbm.at[c, p, pl.ds(s * _TROWS, _TROWS)])

    return scatter_k


def _bd(w):
    z = jnp.zeros_like(w)
    return jnp.concatenate(
        [jnp.concatenate([w, z], axis=1), jnp.concatenate([z, w], axis=1)],
        axis=0)


def _dup(v):
    return jnp.concatenate([v, v]).reshape(1, -1)


_MM = None


def _mean_mat():
    return _bd(jnp.full((_LAT, _LAT), 1.0 / _LAT, jnp.float32))


def _mlp_wts_packed(mlp):
    out = []
    for w, b in mlp:
        out.append(_bd(w))
        out.append(_dup(b))
    return out


def kernel(x, edge_index, e_features, params):
    senders = edge_index[0]
    receivers = edge_index[1]
    pad = _E_PAD - _E
    sp = jnp.concatenate([senders, jnp.zeros((pad,), jnp.int32)])
    rp = jnp.concatenate([receivers, jnp.zeros((pad,), jnp.int32)])
    rs = jnp.concatenate([receivers, jnp.full((pad,), -1, jnp.int32)])
    lidx = jnp.concatenate(
        [jnp.where((rs >= p * _HALF) & (rs < (p + 1) * _HALF),
                   rs - p * _HALF, _HALF).reshape(_NW, _NIDX, 128)
         for p in range(2)],
        axis=1)
    sidx = sp.reshape(_NW, _NIDX, 128)
    ridx = rp.reshape(_NW, _NIDX, 128)
    ew = e_features.shape[1]
    e_pad = jnp.concatenate(
        [e_features, jnp.zeros((pad, ew), jnp.float32)]).reshape(_EP2, 2 * ew)
    x2 = x.reshape(_NP2, 2 * x.shape[1])
    zeros = jnp.zeros((_TROWS, _LAT), jnp.float32)
    mm = _mean_mat()

    g, b = params['enc_node_ln']
    enc_n_wts = _mlp_wts_packed(params['enc_node_mlp']) + [_dup(g), _dup(b),
                                                           mm]
    g, b = params['enc_edge_ln']
    enc_e_wts = _mlp_wts_packed(params['enc_edge_mlp']) + [_dup(g), _dup(b),
                                                           mm]

    h2 = _enc_call(x2, enc_n_wts, _BLK_N)
    e2 = _enc_call(e_pad, enc_e_wts, _BLK_E)

    for layer in params['gnn']:
        h = h2.reshape(_N, _LAT)
        xj, xi = _gather_kernel()(h, sidx, ridx)
        xi2 = xi.reshape(_EP2, 128)
        xj2 = xj.reshape(_EP2, 128)

        (w1, b1), (w2, b2), (w3, b3) = layer['edge_mlp']
        g, bb = layer['edge_ln']
        edge_wts = [_bd(w1[0:_LAT]), _bd(w1[_LAT:2 * _LAT]),
                    _bd(w1[2 * _LAT:3 * _LAT]),
                    _dup(b1), _bd(w2), _dup(b2), _bd(w3), _dup(b3),
                    _dup(g), _dup(bb), mm]
        m2, e2 = _edge_call(xi2, xj2, e2, edge_wts)

        part = _scatter_kernel()(m2.reshape(_E_PAD, _LAT), lidx, zeros)
        part2 = part.reshape(2, 2, _ACC2, 128)

        (w1, b1), (w2, b2), (w3, b3) = layer['node_mlp']
        g, bb = layer['node_ln']
        node_wts = [_bd(w1[0:_LAT]), _bd(w1[_LAT:2 * _LAT]),
                    _dup(b1), _bd(w2), _dup(b2), _bd(w3), _dup(b3),
                    _dup(g), _dup(bb), mm]
        h2 = _node_call(part2, h2, node_wts)

    dec_wts = _mlp_wts_packed(params['dec_mlp'])
    out_dim = params['dec_mlp'][-1][0].shape[1]
    out2 = _dec_call(h2, dec_wts, out_dim)
    return out2.reshape(_N, out_dim)


# speedup vs baseline: 1.9920x; 1.0117x over previous
import functools

import jax
import jax.numpy as jnp
from jax import lax
from jax.experimental import pallas as pl
from jax.experimental.pallas import tpu as pltpu
from jax.experimental.pallas import tpu_sc as plsc

_N = 50000
_E = 800000
_E_PAD = 802816
_NW = 32
_EPW = _E_PAD // _NW
_NIDX = _EPW // 128
_CHUNK = 512
_NCH = _EPW // _CHUNK
_SCCH = 256
_NSCCH = _EPW // _SCCH
_HALF = 25600
_ACC = 25632
_TROWS = _ACC // 16
_LAT = 64

_EP2 = _E_PAD // 2
_NP2 = _N // 2
_ACC2 = _ACC // 2
_BLK_E = 512
_BLK_N = 200


def _dot(a, w):
    return jnp.dot(a, w, preferred_element_type=jnp.float32)


def _ln_packed(y, g, b, mm):
    mu = _dot(y, mm)
    d = y - mu
    v = _dot(d * d, mm)
    return d / jnp.sqrt(v + 1e-5) * g + b


def _enc_node_body(xa_ref, xb_ref, w1l, w1r, b1, w2, b2, w3, b3, g, bb, mm,
                   o_ref):
    y = (_dot(xa_ref[...], w1l[...]) + _dot(xb_ref[...], w1r[...])
         + b1[...])
    y = jnp.maximum(y, 0.0)
    y = jnp.maximum(_dot(y, w2[...]) + b2[...], 0.0)
    y = _dot(y, w3[...]) + b3[...]
    o_ref[...] = _ln_packed(y, g[...], bb[...], mm[...])


def _enc_edge_body(x_ref, w1, b1, w2, b2, w3, b3, g, bb, mm, o_ref):
    y = jnp.maximum(_dot(x_ref[...], w1[...]) + b1[...], 0.0)
    y = jnp.maximum(_dot(y, w2[...]) + b2[...], 0.0)
    y = _dot(y, w3[...]) + b3[...]
    o_ref[...] = _ln_packed(y, g[...], bb[...], mm[...])


def _edge_body(xi, xj, e, w1a, w1b, w1c, b1, w2, b2, w3, b3, g, bb, mm,
               m_o, e_o):
    ev = e[...]
    y = (_dot(xi[...], w1a[...]) + _dot(xj[...], w1b[...])
         + _dot(ev, w1c[...]) + b1[...])
    y = jnp.maximum(y, 0.0)
    y = jnp.maximum(_dot(y, w2[...]) + b2[...], 0.0)
    y = _dot(y, w3[...]) + b3[...]
    m = _ln_packed(y, g[...], bb[...], mm[...])
    m_o[...] = m
    e_o[...] = m + ev


def _node_body(pa, pb, h, w1a, w1b, b1, w2, b2, w3, b3, g, bb, mm, o_ref):
    hv = h[...]
    agg = pa[0, 0] + pb[0, 0]
    y = _dot(agg, w1a[...]) + _dot(hv, w1b[...]) + b1[...]
    y = jnp.maximum(y, 0.0)
    y = jnp.maximum(_dot(y, w2[...]) + b2[...], 0.0)
    y = _dot(y, w3[...]) + b3[...]
    o_ref[...] = _ln_packed(y, g[...], bb[...], mm[...]) + hv


def _dec_body(h, w1, b1, w2, b2, w3, b3, o_ref):
    y = jnp.maximum(_dot(h[...], w1[...]) + b1[...], 0.0)
    y = jnp.maximum(_dot(y, w2[...]) + b2[...], 0.0)
    o_ref[...] = _dot(y, w3[...]) + b3[...]


def _full(shape):
    nd = len(shape)
    return pl.BlockSpec(shape, lambda i: (0,) * nd)


def _rows(blk, width):
    return pl.BlockSpec((blk, width), lambda i: (i, 0))


def _enc_node_call(x, wts):
    nb = _NP2 // 1000
    spec_a = pl.BlockSpec((1000, x.shape[1]), lambda i: (i, 0))
    spec_b = pl.BlockSpec((1000, x.shape[1]), lambda i: (i + nb, 0))
    return pl.pallas_call(
        _enc_node_body,
        grid=(nb,),
        in_specs=[spec_a, spec_b] + [_full(w.shape) for w in wts],
        out_specs=_rows(1000, 128),
        out_shape=jax.ShapeDtypeStruct((_NP2, 128), jnp.float32),
    )(x, x, *wts)


def _enc_edge_call(e_in, wts):
    blk = 784
    return pl.pallas_call(
        _enc_edge_body,
        grid=(_EP2 // blk,),
        in_specs=[_rows(blk, e_in.shape[1])] + [_full(w.shape) for w in wts],
        out_specs=_rows(blk, 128),
        out_shape=jax.ShapeDtypeStruct((_EP2, 128), jnp.float32),
    )(e_in, *wts)


def _edge_call(xi, xj, e, wts):
    return pl.pallas_call(
        _edge_body,
        grid=(_EP2 // _BLK_E,),
        in_specs=[_rows(_BLK_E, 128)] * 3 + [_full(w.shape) for w in wts],
        out_specs=[_rows(_BLK_E, 128)] * 2,
        out_shape=[jax.ShapeDtypeStruct((_EP2, 128), jnp.float32)] * 2,
    )(xi, xj, e, *wts)


def _node_call(part, h, wts):
    nb = _HALF // 2 // _BLK_N
    pspec_a = pl.BlockSpec((1, 1, _BLK_N, 128),
                           lambda i: (0, i // nb, i - (i // nb) * nb, 0))
    pspec_b = pl.BlockSpec((1, 1, _BLK_N, 128),
                           lambda i: (1, i // nb, i - (i // nb) * nb, 0))
    return pl.pallas_call(
        _node_body,
        grid=(_NP2 // _BLK_N,),
        in_specs=[pspec_a, pspec_b, _rows(_BLK_N, 128)]
        + [_full(w.shape) for w in wts],
        out_specs=_rows(_BLK_N, 128),
        out_shape=jax.ShapeDtypeStruct((_NP2, 128), jnp.float32),
    )(part, part, h, *wts)


def _dec_call(h, wts, out_dim):
    return pl.pallas_call(
        _dec_body,
        grid=(_NP2 // _BLK_N,),
        in_specs=[_rows(_BLK_N, 128)] + [_full(w.shape) for w in wts],
        out_specs=_rows(_BLK_N, 2 * out_dim),
        out_shape=jax.ShapeDtypeStruct((_NP2, 2 * out_dim), jnp.float32),
    )(h, *wts)


def _remap_q(r):
    return jnp.where(r >= _NP2, 2 * r - (2 * _NP2 - 1), 2 * r)


@functools.cache
def _gather_kernel():
    mesh = plsc.VectorSubcoreMesh(core_axis_name="c", subcore_axis_name="s")

    @functools.partial(
        pl.kernel,
        out_type=[jax.ShapeDtypeStruct((_E_PAD, _LAT), jnp.float32),
                  jax.ShapeDtypeStruct((_E_PAD, _LAT), jnp.float32)],
        mesh=mesh,
        scratch_types=[pltpu.VMEM((_NIDX, 128), jnp.int32),
                       pltpu.VMEM((_NIDX, 128), jnp.int32),
                       pltpu.VMEM((_CHUNK, _LAT), jnp.float32),
                       pltpu.SemaphoreType.DMA],
        compiler_params=pltpu.CompilerParams(use_tc_tiling_on_sc=False),
    )
    def gather_k(h_hbm, eidx_hbm, xj_hbm, xi_hbm, idx_v, qidx_v, rows_v,
                 sem):
        wid = lax.axis_index("s") * 2 + lax.axis_index("c")
        base = wid * _EPW

        def run(which, out_hbm):
            pltpu.sync_copy(eidx_hbm.at[which, wid], idx_v)

            @pl.loop(0, _NIDX)
            def _remap(row):
                for k in range(8):
                    r = idx_v[row, pl.ds(k * 16, 16)]
                    qidx_v[row, pl.ds(k * 16, 16)] = jnp.maximum(
                        _remap_q(r), 0)

            @pl.loop(0, _NCH)
            def _chunk(i):
                cps = [
                    pltpu.async_copy(h_hbm.at[qidx_v.at[i * 4 + jj]],
                                     rows_v.at[pl.ds(jj * 128, 128)], sem)
                    for jj in range(4)
                ]
                for cp in cps:
                    cp.wait()
                pltpu.sync_copy(rows_v,
                                out_hbm.at[pl.ds(base + i * _CHUNK, _CHUNK)])

        run(0, xj_hbm)
        run(1, xi_hbm)

    return gather_k


@functools.cache
def _scatter_kernel():
    mesh = plsc.VectorSubcoreMesh(core_axis_name="c", subcore_axis_name="s")

    @functools.partial(
        pl.kernel,
        out_type=jax.ShapeDtypeStruct((2, 2, _ACC, _LAT), jnp.float32),
        mesh=mesh,
        scratch_types=[pltpu.VMEM((2, 128), jnp.int32),
                       pltpu.VMEM((2, 128), jnp.int32),
                       pltpu.VMEM((_SCCH, _LAT), jnp.float32),
                       pltpu.VMEM_SHARED((_ACC, _LAT), jnp.float32)],
        compiler_params=pltpu.CompilerParams(use_tc_tiling_on_sc=False),
    )
    def scatter_k(m_hbm, eidx_hbm, zeros_hbm, part_hbm, idx_v, lidx_v,
                  rows_v, acc):
        c = lax.axis_index("c")
        s = lax.axis_index("s")
        wid = s * 2 + c
        base = wid * _EPW
        for p in range(2):
            lo = p * _HALF
            pltpu.sync_copy(zeros_hbm, acc.at[pl.ds(s * _TROWS, _TROWS)])
            plsc.subcore_barrier()

            @pl.loop(0, _NSCCH)
            def _chunk(i):
                pltpu.sync_copy(eidx_hbm.at[1, wid, pl.ds(i * 2, 2)], idx_v)
                pltpu.sync_copy(m_hbm.at[pl.ds(base + i * _SCCH, _SCCH)],
                                rows_v)
                for jj in range(2):
                    for k in range(8):
                        r = idx_v[jj, pl.ds(k * 16, 16)]
                        q = _remap_q(r)
                        ok = (q >= lo) & (q < lo + _HALF)
                        lidx_v[jj, pl.ds(k * 16, 16)] = jnp.where(
                            ok, q - lo, _HALF)
                for jj in range(2):
                    pltpu.sync_copy(
                        rows_v.at[pl.ds(jj * 128, 128)],
                        acc.at[lidx_v.at[jj]],
                        add=True)

            plsc.subcore_barrier()
            pltpu.sync_copy(acc.at[pl.ds(s * _TROWS, _TROWS)],
                            part_hbm.at[c, p, pl.ds(s * _TROWS, _TROWS)])

    return scatter_k


def _bd(w):
    z = jnp.zeros_like(w)
    return jnp.concatenate(
        [jnp.concatenate([w, z], axis=1), jnp.concatenate([z, w], axis=1)],
        axis=0)


def _dup(v):
    return jnp.concatenate([v, v]).reshape(1, -1)


def _mean_mat():
    return _bd(jnp.full((_LAT, _LAT), 1.0 / _LAT, jnp.float32))


def kernel(x, edge_index, e_features, params):
    pad = _E_PAD - _E
    padcol = jnp.concatenate(
        [jnp.zeros((1, pad), jnp.int32), jnp.full((1, pad), -1, jnp.int32)])
    eidx = jnp.concatenate([edge_index, padcol],
                           axis=1).reshape(2, _NW, _NIDX, 128)
    ew = e_features.shape[1]
    e_in = jnp.concatenate(
        [e_features, jnp.zeros((pad, ew), jnp.float32)]).reshape(_EP2, 2 * ew)
    zeros = jnp.zeros((_TROWS, _LAT), jnp.float32)
    mm = _mean_mat()

    (w1, b1), (w2, b2), (w3, b3) = params['enc_node_mlp']
    g, b = params['enc_node_ln']
    zw1 = jnp.zeros_like(w1)
    enc_n_wts = [jnp.concatenate([w1, zw1], axis=1),
                 jnp.concatenate([zw1, w1], axis=1),
                 _dup(b1), _bd(w2), _dup(b2), _bd(w3), _dup(b3),
                 _dup(g), _dup(b), mm]
    g, b = params['enc_edge_ln']
    enc_e_wts = []
    for w, bv in params['enc_edge_mlp']:
        enc_e_wts += [_bd(w), _dup(bv)]
    enc_e_wts += [_dup(g), _dup(b), mm]

    h2 = _enc_node_call(x, enc_n_wts)
    e2 = _enc_edge_call(e_in, enc_e_wts)

    for layer in params['gnn']:
        h = h2.reshape(_N, _LAT)
        xj, xi = _gather_kernel()(h, eidx)
        xi2 = xi.reshape(_EP2, 128)
        xj2 = xj.reshape(_EP2, 128)

        (w1, b1), (w2, b2), (w3, b3) = layer['edge_mlp']
        g, bb = layer['edge_ln']
        edge_wts = [_bd(w1[0:_LAT]), _bd(w1[_LAT:2 * _LAT]),
                    _bd(w1[2 * _LAT:3 * _LAT]),
                    _dup(b1), _bd(w2), _dup(b2), _bd(w3), _dup(b3),
                    _dup(g), _dup(bb), mm]
        m2, e2 = _edge_call(xi2, xj2, e2, edge_wts)

        part = _scatter_kernel()(m2.reshape(_E_PAD, _LAT), eidx, zeros)
        part2 = part.reshape(2, 2, _ACC2, 128)

        (w1, b1), (w2, b2), (w3, b3) = layer['node_mlp']
        g, bb = layer['node_ln']
        node_wts = [_bd(w1[0:_LAT]), _bd(w1[_LAT:2 * _LAT]),
                    _dup(b1), _bd(w2), _dup(b2), _bd(w3), _dup(b3),
                    _dup(g), _dup(bb), mm]
        h2 = _node_call(part2, h2, node_wts)

    dec_wts = []
    for w, bv in params['dec_mlp']:
        dec_wts += [_bd(w), _dup(bv)]
    out_dim = params['dec_mlp'][-1][0].shape[1]
    out2 = _dec_call(h2, dec_wts, out_dim)
    return jnp.concatenate([out2[:, 0:out_dim], out2[:, out_dim:]], axis=0)
